# baseline (speedup 1.0000x reference)
import functools

import jax
import jax.numpy as jnp
from jax import lax
from jax.experimental import pallas as pl
from jax.experimental.pallas import tpu as pltpu
from jax.experimental.pallas import tpu_sc as plsc

H = 8
C = 16
NPAD = 10240
ROWS_PER_TILE = NPAD // 16
CH1 = 128
CH2 = 64
P1 = CH1 // 2
P2 = CH2 // 2


def _node_prepass(x_ref, wsrc_t_ref, u_ref, xs_ref, t_ref):
    xb = x_ref[...]
    xs_ref[...] = jnp.dot(xb, wsrc_t_ref[...], preferred_element_type=jnp.float32)
    t_ref[...] = jnp.dot(xb, u_ref[...], preferred_element_type=jnp.float32)


def _edge_prepass(ea_ref, etf_ref, v_ref, ta_ref, fe_ref):
    oh = (etf_ref[...] ==
          lax.broadcasted_iota(jnp.int32, (1, 7), 1).astype(jnp.float32))
    fe_ref[...] = (
        jnp.dot(ea_ref[...], v_ref[...], preferred_element_type=jnp.float32)
        + jnp.dot(oh.astype(jnp.float32), ta_ref[...],
                  preferred_element_type=jnp.float32))


def _final_dense(p_ref, x_ref, wout_t_ref, bb_ref, g_ref, b_ref, y_ref):
    o = p_ref[0] + p_ref[1]
    o = jnp.dot(o, wout_t_ref[...], preferred_element_type=jnp.float32)
    o = o + bb_ref[...]
    mu = jnp.mean(o, axis=-1, keepdims=True)
    d = o - mu
    var = jnp.mean(d * d, axis=-1, keepdims=True)
    o = d / jnp.sqrt(var + 1e-5) * g_ref[...] + b_ref[...]
    y_ref[...] = o + x_ref[...]


def _bcast(v, k):
    idx = jnp.full((16, 1), k, jnp.int32)
    dn = lax.GatherDimensionNumbers(offset_dims=(), collapsed_slice_dims=(0,),
                                    start_index_map=(0,))
    return lax.gather(v, idx, dn, (1,),
                      mode=lax.GatherScatterMode.PROMISE_IN_BOUNDS)


def _sc_body(nct1, src_h, dst_h, t_h, fe_h, xs_h, outp_h, ex_h,
             s1a, d1a, dc1a, dc1b, tda, tdb, tsa, tsb, fea, feb,
             exa, exb, s2a, d2a, dc2a, dc2b, xra, xrb, e2a, e2b,
             esa, esb, out_sh, es_sh, semia, semib, semga, semgb,
             semla, semlb, semsa, semsb):
    sc = lax.axis_index("c")
    tid = lax.axis_index("s")
    nct2 = 2 * nct1
    g1 = 2 * nct1
    nsc1 = nct1 * 16
    lane = lax.iota(jnp.int32, 16)
    hlane = jnp.bitwise_and(lane, 7)
    half = lax.shift_right_logical(lane, 3)
    zero16 = jnp.zeros((16,), jnp.float32)

    def e0_p1(gi):
        own = gi < nct1
        gid = jnp.where(own, sc * nsc1 + tid * nct1 + gi,
                        (1 - sc) * nsc1 + tid * nct1 + (gi - nct1))
        return gid * CH1, own

    def e0_p2(gi):
        return ((sc * 16 + tid) * nct2 + gi) * CH2

    def z_xr(r, _):
        for c8 in range(8):
            xra[r, pl.ds(c8 * 16, 16)] = zero16
        return 0
    lax.fori_loop(0, CH2, z_xr, 0)

    def z_esd(p, _):
        plsc.store_scatter(esa, [2 * p + half, hlane], zero16)
        return 0
    lax.fori_loop(0, P2, z_esd, 0)

    for j in range(ROWS_PER_TILE // CH2):
        pltpu.sync_copy(xra, out_sh.at[pl.ds(tid * ROWS_PER_TILE + j * CH2, CH2)])
        pltpu.sync_copy(esa, es_sh.at[pl.ds(tid * ROWS_PER_TILE + j * CH2, CH2)])
    plsc.subcore_barrier()

    def p1_compute(ctd, cts, cfe, cex):
        def pair(p, _):
            row = 2 * p + half
            ai = plsc.load_gather(ctd, [row, hlane])
            aj = plsc.load_gather(cts, [row, 8 + hlane])
            fev = cfe[pl.ds(p * 16, 16)]
            lg = ai + aj + fev
            lg = jnp.maximum(lg, 0.2 * lg)
            exv = jnp.exp(lg)
            plsc.store_scatter(cex, [row, hlane], exv)
            return 0
        lax.fori_loop(0, P1, pair, 0)

    def body1(i, _):
        gi = 2 * i
        e0, own = e0_p1(gi)
        pltpu.sync_copy(src_h.at[pl.ds(e0, 2 * CH1)], s1a)
        pltpu.sync_copy(dst_h.at[pl.ds(e0, 2 * CH1)], d1a)
        for q in range(CH1 // 16):
            dc1a[pl.ds(q * 16, 16)] = d1a[pl.ds(q * 16, 16)]
            dc1b[pl.ds(q * 16, 16)] = d1a[pl.ds(CH1 + q * 16, 16)]
        ga1 = pltpu.async_copy(t_h.at[dc1a], tda, semia)
        ga2 = pltpu.async_copy(t_h.at[s1a.at[pl.ds(0, CH1)]], tsa, semga)
        ga3 = pltpu.async_copy(fe_h.at[pl.ds(e0 * 8, CH1 * 8)], fea, semla)
        gb1 = pltpu.async_copy(t_h.at[dc1b], tdb, semib)
        gb2 = pltpu.async_copy(t_h.at[s1a.at[pl.ds(CH1, CH1)]], tsb, semgb)
        gb3 = pltpu.async_copy(fe_h.at[pl.ds((e0 + CH1) * 8, CH1 * 8)], feb,
                               semlb)
        ga1.wait(); ga2.wait(); ga3.wait()
        p1_compute(tda, tsa, fea, exa)
        sca = pltpu.async_copy(exa, es_sh.at[dc1a], semsa, add=True)
        gb1.wait(); gb2.wait(); gb3.wait()
        p1_compute(tdb, tsb, feb, exb)
        scb = pltpu.async_copy(exb, es_sh.at[dc1b], semsb, add=True)
        sca.wait()

        @pl.when(own)
        def _():
            pltpu.sync_copy(exa, ex_h.at[pl.ds(e0, CH1)])
            pltpu.sync_copy(exb, ex_h.at[pl.ds(e0 + CH1, CH1)])
        scb.wait()
        return 0
    lax.fori_loop(0, nct1, body1, 0)
    plsc.subcore_barrier()

    def p2_compute(cxr, ce2, ces):
        def pair(p, _):
            row = 2 * p + half
            exv = plsc.load_gather(ce2, [row, hlane])
            esv = plsc.load_gather(ces, [row, hlane])
            wv = exv / esv
            for e in range(2):
                for h in range(H):
                    s = pl.ds(h * 16, 16)
                    r = 2 * p + e
                    cxr[r, s] = cxr[r, s] * _bcast(wv, e * 8 + h)
            return 0
        lax.fori_loop(0, P2, pair, 0)

    def body2(i, _):
        gi = 2 * i
        e0 = e0_p2(gi)
        pltpu.sync_copy(src_h.at[pl.ds(e0, 2 * CH2)], s2a)
        pltpu.sync_copy(dst_h.at[pl.ds(e0, 2 * CH2)], d2a)
        for q in range(CH2 // 16):
            dc2a[pl.ds(q * 16, 16)] = d2a[pl.ds(q * 16, 16)]
            dc2b[pl.ds(q * 16, 16)] = d2a[pl.ds(CH2 + q * 16, 16)]
        ga1 = pltpu.async_copy(xs_h.at[s2a.at[pl.ds(0, CH2)]], xra, semia)
        ga2 = pltpu.async_copy(es_sh.at[dc2a], esa, semga)
        ga3 = pltpu.async_copy(ex_h.at[pl.ds(e0, CH2)], e2a, semla)
        gb1 = pltpu.async_copy(xs_h.at[s2a.at[pl.ds(CH2, CH2)]], xrb, semib)
        gb2 = pltpu.async_copy(es_sh.at[dc2b], esb, semgb)
        gb3 = pltpu.async_copy(ex_h.at[pl.ds(e0 + CH2, CH2)], e2b, semlb)
        ga1.wait(); ga2.wait(); ga3.wait()
        p2_compute(xra, e2a, esa)
        sca = pltpu.async_copy(xra, out_sh.at[dc2a], semsa, add=True)
        gb1.wait(); gb2.wait(); gb3.wait()
        p2_compute(xrb, e2b, esb)
        scb = pltpu.async_copy(xrb, out_sh.at[dc2b], semsb, add=True)
        sca.wait()
        scb.wait()
        return 0
    lax.fori_loop(0, nct1, body2, 0)

    plsc.subcore_barrier()
    pltpu.sync_copy(out_sh.at[pl.ds(tid * ROWS_PER_TILE, ROWS_PER_TILE)],
                    outp_h.at[sc, pl.ds(tid * ROWS_PER_TILE, ROWS_PER_TILE)])


def kernel(x, edge_index, edge_attr, edge_types, W_src, W_dst, att_src,
           att_dst, W_edge, att_edge, edge_type_table, W_out, b_out, bias,
           ln_g, ln_b):
    n, d = x.shape
    e = edge_index.shape[1]
    nt = edge_type_table.shape[0]
    ed = edge_attr.shape[1]
    etot = e + n
    ep = ((etot + 64 * CH1 - 1) // (64 * CH1)) * (64 * CH1)
    nct1 = ep // (32 * CH1)

    u_dst = jnp.einsum('hcd,hc->dh', W_dst.reshape(H, C, d), att_src[0])
    u_src = jnp.einsum('hcd,hc->dh', W_src.reshape(H, C, d), att_dst[0])
    u = jnp.concatenate([u_dst, u_src], axis=1)
    v = jnp.einsum('hce,hc->eh', W_edge.reshape(H, C, ed), att_edge[0])
    ta_vec = jnp.einsum('thc,hc->th', edge_type_table.reshape(nt, H, C),
                        att_edge[0])

    xp = jnp.concatenate([x, jnp.zeros((NPAD - n, d), jnp.float32)])

    nb = NPAD // 256
    xs_p, t_p = pl.pallas_call(
        _node_prepass,
        grid=(nb,),
        in_specs=[pl.BlockSpec((256, d), lambda i: (i, 0)),
                  pl.BlockSpec((d, d), lambda i: (0, 0)),
                  pl.BlockSpec((d, 16), lambda i: (0, 0))],
        out_specs=[pl.BlockSpec((256, d), lambda i: (i, 0)),
                   pl.BlockSpec((256, 16), lambda i: (i, 0))],
        out_shape=[jax.ShapeDtypeStruct((NPAD, d), jnp.float32),
                   jax.ShapeDtypeStruct((NPAD, 16), jnp.float32)],
    )(xp, W_src.T, u)

    eb = 2000
    fe_real = pl.pallas_call(
        _edge_prepass,
        grid=(e // eb,),
        in_specs=[pl.BlockSpec((eb, ed), lambda i: (i, 0)),
                  pl.BlockSpec((eb, 1), lambda i: (i, 0)),
                  pl.BlockSpec((ed, H), lambda i: (0, 0)),
                  pl.BlockSpec((nt, H), lambda i: (0, 0))],
        out_specs=pl.BlockSpec((eb, H), lambda i: (i, 0)),
        out_shape=jax.ShapeDtypeStruct((e, H), jnp.float32),
    )(edge_attr, edge_types.astype(jnp.float32).reshape(e, 1), v, ta_vec)

    loop_idx = jnp.arange(n, dtype=jnp.int32)
    pad_i = jnp.full((ep - etot,), n, jnp.int32)
    src_full = jnp.concatenate([edge_index[0].astype(jnp.int32), loop_idx, pad_i])
    dst_full = jnp.concatenate([edge_index[1].astype(jnp.int32), loop_idx, pad_i])
    fe_loop = v.sum(0) + ta_vec[nt - 1]
    fe_full = jnp.concatenate([
        fe_real.reshape(-1),
        jnp.broadcast_to(fe_loop, (n, H)).reshape(-1),
        jnp.zeros(((ep - etot) * H,), jnp.float32)])

    mesh = plsc.VectorSubcoreMesh(core_axis_name="c", subcore_axis_name="s")
    outp, _ex = pl.kernel(
        functools.partial(_sc_body, nct1),
        out_type=[jax.ShapeDtypeStruct((2, NPAD, d), jnp.float32),
                  jax.ShapeDtypeStruct((ep, H), jnp.float32)],
        mesh=mesh,
        compiler_params=pltpu.CompilerParams(needs_layout_passes=False,
                                             use_tc_tiling_on_sc=False),
        scratch_types=[
            pltpu.VMEM((2 * CH1,), jnp.int32),
            pltpu.VMEM((2 * CH1,), jnp.int32),
            pltpu.VMEM((CH1,), jnp.int32),
            pltpu.VMEM((CH1,), jnp.int32),
            pltpu.VMEM((CH1, 16), jnp.float32),
            pltpu.VMEM((CH1, 16), jnp.float32),
            pltpu.VMEM((CH1, 16), jnp.float32),
            pltpu.VMEM((CH1, 16), jnp.float32),
            pltpu.VMEM((CH1 * 8,), jnp.float32),
            pltpu.VMEM((CH1 * 8,), jnp.float32),
            pltpu.VMEM((CH1, 8), jnp.float32),
            pltpu.VMEM((CH1, 8), jnp.float32),
            pltpu.VMEM((2 * CH2,), jnp.int32),
            pltpu.VMEM((2 * CH2,), jnp.int32),
            pltpu.VMEM((CH2,), jnp.int32),
            pltpu.VMEM((CH2,), jnp.int32),
            pltpu.VMEM((CH2, 128), jnp.float32),
            pltpu.VMEM((CH2, 128), jnp.float32),
            pltpu.VMEM((CH2, 8), jnp.float32),
            pltpu.VMEM((CH2, 8), jnp.float32),
            pltpu.VMEM((CH2, 8), jnp.float32),
            pltpu.VMEM((CH2, 8), jnp.float32),
            pltpu.VMEM_SHARED((NPAD, 128), jnp.float32),
            pltpu.VMEM_SHARED((NPAD, 8), jnp.float32),
            pltpu.SemaphoreType.DMA,
            pltpu.SemaphoreType.DMA,
            pltpu.SemaphoreType.DMA,
            pltpu.SemaphoreType.DMA,
            pltpu.SemaphoreType.DMA,
            pltpu.SemaphoreType.DMA,
            pltpu.SemaphoreType.DMA,
            pltpu.SemaphoreType.DMA,
        ],
    )(src_full, dst_full, t_p, fe_full, xs_p)

    y = pl.pallas_call(
        _final_dense,
        grid=(nb,),
        in_specs=[pl.BlockSpec((2, 256, d), lambda i: (0, i, 0)),
                  pl.BlockSpec((256, d), lambda i: (i, 0)),
                  pl.BlockSpec((d, d), lambda i: (0, 0)),
                  pl.BlockSpec((1, d), lambda i: (0, 0)),
                  pl.BlockSpec((1, d), lambda i: (0, 0)),
                  pl.BlockSpec((1, d), lambda i: (0, 0))],
        out_specs=pl.BlockSpec((256, d), lambda i: (i, 0)),
        out_shape=jax.ShapeDtypeStruct((NPAD, d), jnp.float32),
    )(outp, xp, W_out.T, (b_out + bias).reshape(1, d),
      ln_g.reshape(1, d), ln_b.reshape(1, d))
    return y[:n]


# speedup vs baseline: 1.3471x; 1.3471x over previous
import functools

import jax
import jax.numpy as jnp
from jax import lax
from jax.experimental import pallas as pl
from jax.experimental.pallas import tpu as pltpu
from jax.experimental.pallas import tpu_sc as plsc

H = 8
C = 16
NPAD = 10240
ROWS_PER_TILE = NPAD // 16
CH = 128
PAIRS = CH // 2


def _node_prepass(x_ref, wsrc_t_ref, u_ref, xs_ref, t_ref):
    xb = x_ref[...]
    xs_ref[...] = jnp.dot(xb, wsrc_t_ref[...], preferred_element_type=jnp.float32)
    t_ref[...] = jnp.dot(xb, u_ref[...], preferred_element_type=jnp.float32)


def _edge_prepass(ea_ref, etf_ref, v_ref, ta_ref, fe_ref):
    oh = (etf_ref[...] ==
          lax.broadcasted_iota(jnp.int32, (1, 7), 1).astype(jnp.float32))
    fe_ref[...] = (
        jnp.dot(ea_ref[...], v_ref[...], preferred_element_type=jnp.float32)
        + jnp.dot(oh.astype(jnp.float32), ta_ref[...],
                  preferred_element_type=jnp.float32))


def _final_dense(p_ref, esx_ref, x_ref, wout_t_ref, bb_ref, g_ref, b_ref,
                 y_ref):
    o = (p_ref[0] + p_ref[1]) / esx_ref[...]
    o = jnp.dot(o, wout_t_ref[...], preferred_element_type=jnp.float32)
    o = o + bb_ref[...]
    mu = jnp.mean(o, axis=-1, keepdims=True)
    d = o - mu
    var = jnp.mean(d * d, axis=-1, keepdims=True)
    o = d / jnp.sqrt(var + 1e-5) * g_ref[...] + b_ref[...]
    y_ref[...] = o + x_ref[...]


def _bcast(v, k):
    idx = jnp.full((16, 1), k, jnp.int32)
    dn = lax.GatherDimensionNumbers(offset_dims=(), collapsed_slice_dims=(0,),
                                    start_index_map=(0,))
    return lax.gather(v, idx, dn, (1,),
                      mode=lax.GatherScatterMode.PROMISE_IN_BOUNDS)


def _sc_body(nct, src_h, dst_h, t_h, fe_h, xs_h, outp_h, eso_h,
             s1, d1, dca, dcb, tda, tdb, tsa, tsb, fea, feb, exa, exb,
             xra, xrb, out_sh, es_sh,
             mta, mtb, msa, msb, mfa, mfb, mxa, mxb, mea, meb, moa, mob):
    sc = lax.axis_index("c")
    tid = lax.axis_index("s")
    lane = lax.iota(jnp.int32, 16)
    hlane = jnp.bitwise_and(lane, 7)
    half = lax.shift_right_logical(lane, 3)
    zero16 = jnp.zeros((16,), jnp.float32)
    hi_mask = jnp.full((16,), 0xFFFF0000, jnp.uint32)
    base = (sc * 16 + tid) * nct

    def z_xr(r, _):
        for c8 in range(8):
            xra[r, pl.ds(c8 * 16, 16)] = zero16
        return 0
    lax.fori_loop(0, CH, z_xr, 0)

    def z_ex(p, _):
        plsc.store_scatter(exa, [2 * p + half, hlane], zero16)
        return 0
    lax.fori_loop(0, PAIRS, z_ex, 0)

    for j in range(ROWS_PER_TILE // CH):
        r0 = tid * ROWS_PER_TILE + j * CH
        pltpu.sync_copy(xra, out_sh.at[pl.ds(r0, CH)])
        pltpu.sync_copy(exa, es_sh.at[pl.ds(r0, CH)])
    plsc.subcore_barrier()

    def pairs_ex(td, ts, fe, ex):
        def pair(p, _):
            row = 2 * p + half
            wd = plsc.bitcast(plsc.load_gather(td, [row, hlane]), jnp.uint32)
            ws = plsc.bitcast(plsc.load_gather(ts, [row, hlane]), jnp.uint32)
            ai = plsc.bitcast(lax.shift_left(wd, jnp.uint32(16)), jnp.float32)
            aj = plsc.bitcast(jnp.bitwise_and(ws, hi_mask), jnp.float32)
            fev = fe[pl.ds(p * 16, 16)]
            lg = ai + aj + fev
            lg = jnp.maximum(lg, 0.2 * lg)
            exv = jnp.exp(lg)
            plsc.store_scatter(ex, [row, hlane], exv)
            return 0
        lax.fori_loop(0, PAIRS, pair, 0)

    def scale(xr, ex):
        def pair(p, _):
            row = 2 * p + half
            exv = plsc.load_gather(ex, [row, hlane])
            for e in range(2):
                for h in range(H):
                    s = pl.ds(h * 16, 16)
                    r = 2 * p + e
                    xr[r, s] = xr[r, s] * _bcast(exv, e * 8 + h)
            return 0
        lax.fori_loop(0, PAIRS, pair, 0)

    def body(i, _):
        e0 = (base + 2 * i) * CH
        pltpu.sync_copy(src_h.at[pl.ds(e0, 2 * CH)], s1)
        pltpu.sync_copy(dst_h.at[pl.ds(e0, 2 * CH)], d1)
        for q in range(CH // 16):
            dca[pl.ds(q * 16, 16)] = d1[pl.ds(q * 16, 16)]
            dcb[pl.ds(q * 16, 16)] = d1[pl.ds(CH + q * 16, 16)]
        sa = s1.at[pl.ds(0, CH)]
        sb = s1.at[pl.ds(CH, CH)]
        gxa = pltpu.async_copy(xs_h.at[sa], xra, mxa)
        gxb = pltpu.async_copy(xs_h.at[sb], xrb, mxb)
        gta = pltpu.async_copy(t_h.at[dca], tda, mta)
        gsa = pltpu.async_copy(t_h.at[sa], tsa, msa)
        gfa = pltpu.async_copy(fe_h.at[pl.ds(e0 * 8, CH * 8)], fea, mfa)
        gtb = pltpu.async_copy(t_h.at[dcb], tdb, mtb)
        gsb = pltpu.async_copy(t_h.at[sb], tsb, msb)
        gfb = pltpu.async_copy(fe_h.at[pl.ds((e0 + CH) * 8, CH * 8)], feb, mfb)
        gta.wait(); gsa.wait(); gfa.wait()
        pairs_ex(tda, tsa, fea, exa)
        sea = pltpu.async_copy(exa, es_sh.at[dca], mea, add=True)
        gxa.wait()
        scale(xra, exa)
        soa = pltpu.async_copy(xra, out_sh.at[dca], moa, add=True)
        gtb.wait(); gsb.wait(); gfb.wait()
        pairs_ex(tdb, tsb, feb, exb)
        seb = pltpu.async_copy(exb, es_sh.at[dcb], meb, add=True)
        gxb.wait()
        scale(xrb, exb)
        sob = pltpu.async_copy(xrb, out_sh.at[dcb], mob, add=True)
        sea.wait(); soa.wait(); seb.wait(); sob.wait()
        return 0
    lax.fori_loop(0, nct // 2, body, 0)

    plsc.subcore_barrier()
    r0 = tid * ROWS_PER_TILE
    pltpu.sync_copy(out_sh.at[pl.ds(r0, ROWS_PER_TILE)],
                    outp_h.at[sc, pl.ds(r0, ROWS_PER_TILE)])
    pltpu.sync_copy(es_sh.at[pl.ds(r0, ROWS_PER_TILE)],
                    eso_h.at[sc, pl.ds(r0, ROWS_PER_TILE)])


def kernel(x, edge_index, edge_attr, edge_types, W_src, W_dst, att_src,
           att_dst, W_edge, att_edge, edge_type_table, W_out, b_out, bias,
           ln_g, ln_b):
    n, d = x.shape
    e = edge_index.shape[1]
    nt = edge_type_table.shape[0]
    ed = edge_attr.shape[1]
    etot = e + n
    ep = ((etot + 64 * CH - 1) // (64 * CH)) * (64 * CH)
    nct = ep // (32 * CH)

    u_dst = jnp.einsum('hcd,hc->dh', W_dst.reshape(H, C, d), att_src[0])
    u_src = jnp.einsum('hcd,hc->dh', W_src.reshape(H, C, d), att_dst[0])
    u = jnp.concatenate([u_dst, u_src], axis=1)
    v = jnp.einsum('hce,hc->eh', W_edge.reshape(H, C, ed), att_edge[0])
    ta_vec = jnp.einsum('thc,hc->th', edge_type_table.reshape(nt, H, C),
                        att_edge[0])

    xp = jnp.concatenate([x, jnp.zeros((NPAD - n, d), jnp.float32)])

    nb = NPAD // 256
    xs_p, t_p = pl.pallas_call(
        _node_prepass,
        grid=(nb,),
        in_specs=[pl.BlockSpec((256, d), lambda i: (i, 0)),
                  pl.BlockSpec((d, d), lambda i: (0, 0)),
                  pl.BlockSpec((d, 16), lambda i: (0, 0))],
        out_specs=[pl.BlockSpec((256, d), lambda i: (i, 0)),
                   pl.BlockSpec((256, 16), lambda i: (i, 0))],
        out_shape=[jax.ShapeDtypeStruct((NPAD, d), jnp.float32),
                   jax.ShapeDtypeStruct((NPAD, 16), jnp.float32)],
    )(xp, W_src.T, u)

    ai16 = lax.bitcast_convert_type(t_p[:, :8].astype(jnp.bfloat16),
                                    jnp.uint16).astype(jnp.uint32)
    aj16 = lax.bitcast_convert_type(t_p[:, 8:].astype(jnp.bfloat16),
                                    jnp.uint16).astype(jnp.uint32)
    t32 = lax.bitcast_convert_type(ai16 | (aj16 << 16), jnp.float32)

    eb = 2000
    fe_real = pl.pallas_call(
        _edge_prepass,
        grid=(e // eb,),
        in_specs=[pl.BlockSpec((eb, ed), lambda i: (i, 0)),
                  pl.BlockSpec((eb, 1), lambda i: (i, 0)),
                  pl.BlockSpec((ed, H), lambda i: (0, 0)),
                  pl.BlockSpec((nt, H), lambda i: (0, 0))],
        out_specs=pl.BlockSpec((eb, H), lambda i: (i, 0)),
        out_shape=jax.ShapeDtypeStruct((e, H), jnp.float32),
    )(edge_attr, edge_types.astype(jnp.float32).reshape(e, 1), v, ta_vec)

    loop_idx = jnp.arange(n, dtype=jnp.int32)
    pad_i = jnp.full((ep - etot,), n, jnp.int32)
    src_full = jnp.concatenate([edge_index[0].astype(jnp.int32), loop_idx, pad_i])
    dst_full = jnp.concatenate([edge_index[1].astype(jnp.int32), loop_idx, pad_i])
    fe_loop = v.sum(0) + ta_vec[nt - 1]
    fe_full = jnp.concatenate([
        fe_real.reshape(-1),
        jnp.broadcast_to(fe_loop, (n, H)).reshape(-1),
        jnp.zeros(((ep - etot) * H,), jnp.float32)])

    mesh = plsc.VectorSubcoreMesh(core_axis_name="c", subcore_axis_name="s")
    outp, eso = pl.kernel(
        functools.partial(_sc_body, nct),
        out_type=[jax.ShapeDtypeStruct((2, NPAD, d), jnp.float32),
                  jax.ShapeDtypeStruct((2, NPAD, H), jnp.float32)],
        mesh=mesh,
        compiler_params=pltpu.CompilerParams(needs_layout_passes=False,
                                             use_tc_tiling_on_sc=False),
        scratch_types=[
            pltpu.VMEM((2 * CH,), jnp.int32),
            pltpu.VMEM((2 * CH,), jnp.int32),
            pltpu.VMEM((CH,), jnp.int32),
            pltpu.VMEM((CH,), jnp.int32),
            pltpu.VMEM((CH, 8), jnp.float32),
            pltpu.VMEM((CH, 8), jnp.float32),
            pltpu.VMEM((CH, 8), jnp.float32),
            pltpu.VMEM((CH, 8), jnp.float32),
            pltpu.VMEM((CH * 8,), jnp.float32),
            pltpu.VMEM((CH * 8,), jnp.float32),
            pltpu.VMEM((CH, 8), jnp.float32),
            pltpu.VMEM((CH, 8), jnp.float32),
            pltpu.VMEM((CH, 128), jnp.float32),
            pltpu.VMEM((CH, 128), jnp.float32),
            pltpu.VMEM_SHARED((NPAD, 128), jnp.float32),
            pltpu.VMEM_SHARED((NPAD, 8), jnp.float32),
        ] + [pltpu.SemaphoreType.DMA] * 12,
    )(src_full, dst_full, t32, fe_full, xs_p)

    esx = jnp.repeat(eso[0] + eso[1], C, axis=1) + 1e-16
    y = pl.pallas_call(
        _final_dense,
        grid=(nb,),
        in_specs=[pl.BlockSpec((2, 256, d), lambda i: (0, i, 0)),
                  pl.BlockSpec((256, d), lambda i: (i, 0)),
                  pl.BlockSpec((256, d), lambda i: (i, 0)),
                  pl.BlockSpec((d, d), lambda i: (0, 0)),
                  pl.BlockSpec((1, d), lambda i: (0, 0)),
                  pl.BlockSpec((1, d), lambda i: (0, 0)),
                  pl.BlockSpec((1, d), lambda i: (0, 0))],
        out_specs=pl.BlockSpec((256, d), lambda i: (i, 0)),
        out_shape=jax.ShapeDtypeStruct((NPAD, d), jnp.float32),
    )(outp, esx, xp, W_out.T, (b_out + bias).reshape(1, d),
      ln_g.reshape(1, d), ln_b.reshape(1, d))
    return y[:n]
